# Initial kernel scaffold; baseline (speedup 1.0000x reference)
import jax
import jax.numpy as jnp
from jax.experimental import pallas as pl


def kernel(hidden_states, expert_affinities, expert_index, w_gate, w_up, w_down, seq_len):
    raise NotImplementedError("write your pallas kernel here")


# speedup vs baseline: 1.1920x; 1.1920x over previous
import functools

import jax
import jax.numpy as jnp
from jax.experimental import pallas as pl
from jax.experimental.pallas import tpu as pltpu

T, H, I, E, K = 2048, 1024, 2048, 8, 2
TK = T * K
BLK = 128
NROWS = TK + E * BLK
NB = NROWS // BLK


def _routing(expert_affinities, expert_index):
    e_flat = expert_index.reshape(TK)
    order = jnp.argsort(e_flat, stable=True)
    e_sorted = e_flat[order]
    counts = jnp.zeros((E,), jnp.int32).at[e_flat].add(1)
    padded = ((counts + BLK - 1) // BLK) * BLK
    cum_padded = jnp.cumsum(padded)
    start_pad = cum_padded - padded
    cum_counts = jnp.cumsum(counts)
    start_raw = cum_counts - counts
    j = jnp.arange(TK, dtype=jnp.int32)
    dest = start_pad[e_sorted] + (j - start_raw[e_sorted])

    sel_aff = jnp.take_along_axis(expert_affinities, expert_index, axis=1)
    sel_aff = sel_aff / jnp.sum(sel_aff, axis=1, keepdims=True)
    w_flat = sel_aff.reshape(TK)

    row_token = jnp.zeros((NROWS,), jnp.int32).at[dest].set(order // K)
    row_weight = jnp.zeros((NROWS,), jnp.float32).at[dest].set(w_flat[order])
    pair_pos = jnp.zeros((TK,), jnp.int32).at[order].set(dest).reshape(T, K)
    block_expert = jnp.clip(
        jnp.searchsorted(cum_padded, jnp.arange(NB, dtype=jnp.int32) * BLK,
                         side="right"),
        0, E - 1).astype(jnp.int32)
    return row_token, row_weight, block_expert, pair_pos


def _mlp_block(be_ref, x_ref, wg_ref, wu_ref, wd_ref, wt_ref, o_ref):
    x = x_ref[...]
    gate = jnp.dot(x, wg_ref[0], preferred_element_type=jnp.float32)
    up = jnp.dot(x, wu_ref[0], preferred_element_type=jnp.float32)
    act = gate * jax.nn.sigmoid(gate) * up
    out = jnp.dot(act, wd_ref[0], preferred_element_type=jnp.float32)
    o_ref[...] = out * wt_ref[...]


def _grouped_mlp(xs, w_gate, w_up, w_down, row_weight, block_expert):
    grid_spec = pltpu.PrefetchScalarGridSpec(
        num_scalar_prefetch=1,
        grid=(NB,),
        in_specs=[
            pl.BlockSpec((BLK, H), lambda i, be: (i, 0)),
            pl.BlockSpec((1, H, I), lambda i, be: (be[i], 0, 0)),
            pl.BlockSpec((1, H, I), lambda i, be: (be[i], 0, 0)),
            pl.BlockSpec((1, I, H), lambda i, be: (be[i], 0, 0)),
            pl.BlockSpec((BLK, 1), lambda i, be: (i, 0)),
        ],
        out_specs=pl.BlockSpec((BLK, H), lambda i, be: (i, 0)),
    )
    return pl.pallas_call(
        _mlp_block,
        grid_spec=grid_spec,
        out_shape=jax.ShapeDtypeStruct((NROWS, H), jnp.float32),
    )(block_expert, xs, w_gate, w_up, w_down, row_weight.reshape(NROWS, 1))


def kernel(hidden_states, expert_affinities, expert_index, w_gate, w_up,
           w_down, seq_len):
    row_token, row_weight, block_expert, pair_pos = _routing(
        expert_affinities, expert_index)
    xs = jnp.take(hidden_states, row_token, axis=0)
    ys = _grouped_mlp(xs, w_gate, w_up, w_down, row_weight, block_expert)
    out = jnp.take(ys, pair_pos[:, 0], axis=0) + jnp.take(
        ys, pair_pos[:, 1], axis=0)
    return out


# speedup vs baseline: 1.3183x; 1.1059x over previous
import functools

import jax
import jax.numpy as jnp
from jax.experimental import pallas as pl
from jax.experimental.pallas import tpu as pltpu

T, H, I, E, K = 2048, 1024, 2048, 8, 2
TK = T * K
BLK = 128
NROWS = TK + E * BLK
NB = NROWS // BLK


def _routing(expert_affinities, expert_index):
    e_flat = expert_index.reshape(TK)
    onehot = (e_flat[:, None] == jnp.arange(E, dtype=jnp.int32)[None, :])
    prefix = jnp.cumsum(onehot.astype(jnp.int32), axis=0)
    counts = prefix[-1]
    rank = jnp.take_along_axis(prefix, e_flat[:, None], axis=1)[:, 0] - 1
    padded = ((counts + BLK - 1) // BLK) * BLK
    cum_padded = jnp.cumsum(padded)
    start_pad = cum_padded - padded
    dest = start_pad[e_flat] + rank

    sel_aff = jnp.take_along_axis(expert_affinities, expert_index, axis=1)
    sel_aff = sel_aff / jnp.sum(sel_aff, axis=1, keepdims=True)
    w_flat = sel_aff.reshape(TK)

    tok = jnp.arange(TK, dtype=jnp.int32) // K
    row_token = jnp.zeros((NROWS,), jnp.int32).at[dest].set(tok)
    row_weight = jnp.zeros((NROWS,), jnp.float32).at[dest].set(w_flat)
    pair_pos = dest.reshape(T, K)
    block_expert = jnp.clip(
        jnp.searchsorted(cum_padded, jnp.arange(NB, dtype=jnp.int32) * BLK,
                         side="right"),
        0, E - 1).astype(jnp.int32)
    num_blocks = (cum_padded[-1] // BLK).astype(jnp.int32)
    return row_token, row_weight, block_expert, pair_pos, num_blocks


def _mlp_block(be_ref, x_ref, wg_ref, wu_ref, wd_ref, wt_ref, o_ref):
    @pl.when(pl.program_id(0) < be_ref[NB])
    def _():
        x = x_ref[...].astype(jnp.bfloat16)
        gate = jnp.dot(x, wg_ref[0].astype(jnp.bfloat16),
                       preferred_element_type=jnp.float32)
        up = jnp.dot(x, wu_ref[0].astype(jnp.bfloat16),
                     preferred_element_type=jnp.float32)
        act = (gate * jax.nn.sigmoid(gate) * up).astype(jnp.bfloat16)
        out = jnp.dot(act, wd_ref[0].astype(jnp.bfloat16),
                      preferred_element_type=jnp.float32)
        o_ref[...] = out * wt_ref[...]


def _grouped_mlp(xs, w_gate, w_up, w_down, row_weight, scalars):
    grid_spec = pltpu.PrefetchScalarGridSpec(
        num_scalar_prefetch=1,
        grid=(NB,),
        in_specs=[
            pl.BlockSpec((BLK, H), lambda i, be: (i, 0)),
            pl.BlockSpec((1, H, I), lambda i, be: (be[i], 0, 0)),
            pl.BlockSpec((1, H, I), lambda i, be: (be[i], 0, 0)),
            pl.BlockSpec((1, I, H), lambda i, be: (be[i], 0, 0)),
            pl.BlockSpec((BLK, 1), lambda i, be: (i, 0)),
        ],
        out_specs=pl.BlockSpec((BLK, H), lambda i, be: (i, 0)),
    )
    return pl.pallas_call(
        _mlp_block,
        grid_spec=grid_spec,
        out_shape=jax.ShapeDtypeStruct((NROWS, H), jnp.float32),
    )(scalars, xs, w_gate, w_up, w_down, row_weight.reshape(NROWS, 1))


def kernel(hidden_states, expert_affinities, expert_index, w_gate, w_up,
           w_down, seq_len):
    row_token, row_weight, block_expert, pair_pos, num_blocks = _routing(
        expert_affinities, expert_index)
    scalars = jnp.concatenate([block_expert, num_blocks[None]])
    xs = jnp.take(hidden_states, row_token, axis=0)
    ys = _grouped_mlp(xs, w_gate, w_up, w_down, row_weight, scalars)
    out = jnp.take(ys, pair_pos[:, 0], axis=0) + jnp.take(
        ys, pair_pos[:, 1], axis=0)
    return out


# speedup vs baseline: 1.4571x; 1.1053x over previous
import functools

import jax
import jax.numpy as jnp
from jax import lax
from jax.experimental import pallas as pl
from jax.experimental.pallas import tpu as pltpu
from jax.experimental.pallas import tpu_sc as plsc

T, H, I, E, K = 2048, 1024, 2048, 8, 2
TK = T * K
BLK = 128
NROWS = TK + E * BLK
NB = NROWS // BLK


def _routing(expert_affinities, expert_index):
    e_flat = expert_index.reshape(TK)
    onehot = (e_flat[:, None] == jnp.arange(E, dtype=jnp.int32)[None, :])
    prefix = jnp.cumsum(onehot.astype(jnp.int32), axis=0)
    counts = prefix[-1]
    rank = jnp.take_along_axis(prefix, e_flat[:, None], axis=1)[:, 0] - 1
    padded = ((counts + BLK - 1) // BLK) * BLK
    cum_padded = jnp.cumsum(padded)
    start_pad = cum_padded - padded
    dest = start_pad[e_flat] + rank

    sel_aff = jnp.take_along_axis(expert_affinities, expert_index, axis=1)
    sel_aff = sel_aff / jnp.sum(sel_aff, axis=1, keepdims=True)
    w_flat = sel_aff.reshape(TK)

    tok = jnp.arange(TK, dtype=jnp.int32) // K
    row_token = jnp.zeros((NROWS,), jnp.int32).at[dest].set(tok)
    row_weight = jnp.zeros((NROWS,), jnp.float32).at[dest].set(w_flat)
    pair_pos = dest.reshape(T, K)
    block_expert = jnp.clip(
        jnp.searchsorted(cum_padded, jnp.arange(NB, dtype=jnp.int32) * BLK,
                         side="right"),
        0, E - 1).astype(jnp.int32)
    num_blocks = (cum_padded[-1] // BLK).astype(jnp.int32)
    return row_token, row_weight, block_expert, pair_pos, num_blocks


def _mlp_block(be_ref, x_ref, wg_ref, wu_ref, wd_ref, wt_ref, o_ref):
    @pl.when(pl.program_id(0) < be_ref[NB])
    def _():
        x = x_ref[...].astype(jnp.bfloat16)
        gate = jnp.dot(x, wg_ref[0].astype(jnp.bfloat16),
                       preferred_element_type=jnp.float32)
        up = jnp.dot(x, wu_ref[0].astype(jnp.bfloat16),
                     preferred_element_type=jnp.float32)
        act = (gate * jax.nn.sigmoid(gate) * up).astype(jnp.bfloat16)
        out = jnp.dot(act, wd_ref[0].astype(jnp.bfloat16),
                      preferred_element_type=jnp.float32)
        o_ref[...] = out * wt_ref[...]


def _grouped_mlp(xs, w_gate, w_up, w_down, row_weight, scalars):
    grid_spec = pltpu.PrefetchScalarGridSpec(
        num_scalar_prefetch=1,
        grid=(NB,),
        in_specs=[
            pl.BlockSpec((BLK, H), lambda i, be: (i, 0)),
            pl.BlockSpec((1, H, I), lambda i, be: (be[i], 0, 0)),
            pl.BlockSpec((1, H, I), lambda i, be: (be[i], 0, 0)),
            pl.BlockSpec((1, I, H), lambda i, be: (be[i], 0, 0)),
            pl.BlockSpec((BLK, 1), lambda i, be: (i, 0)),
        ],
        out_specs=pl.BlockSpec((BLK, H), lambda i, be: (i, 0)),
    )
    return pl.pallas_call(
        _mlp_block,
        grid_spec=grid_spec,
        out_shape=jax.ShapeDtypeStruct((NROWS, H), jnp.float32),
    )(scalars, xs, w_gate, w_up, w_down, row_weight.reshape(NROWS, 1))


_SC_MESH = plsc.VectorSubcoreMesh(core_axis_name="c", subcore_axis_name="s")
_NW = 32
_DISP_RPW = NROWS // _NW
_DISP_CH = _DISP_RPW // 2
_COMB_TPW = T // _NW
_COMB_CH = _COMB_TPW // 2


def _sc_wid():
    return lax.axis_index("s") * 2 + lax.axis_index("c")


@functools.partial(
    pl.kernel, mesh=_SC_MESH,
    out_type=jax.ShapeDtypeStruct((NROWS, H), jnp.float32),
    scratch_types=[
        pltpu.VMEM((_DISP_CH,), jnp.int32),
        pltpu.VMEM((_DISP_CH,), jnp.int32),
        pltpu.VMEM((_DISP_CH, H), jnp.float32),
        pltpu.SemaphoreType.DMA,
    ],
)
def _sc_dispatch(hidden_hbm, tok_hbm, xs_hbm, idx_a, idx_b, rows_v, sem):
    base = _sc_wid() * _DISP_RPW
    pltpu.sync_copy(tok_hbm.at[pl.ds(base, _DISP_CH)], idx_a)
    pltpu.sync_copy(tok_hbm.at[pl.ds(base + _DISP_CH, _DISP_CH)], idx_b)
    pltpu.async_copy(hidden_hbm.at[idx_a], rows_v, sem).wait()
    pltpu.sync_copy(rows_v, xs_hbm.at[pl.ds(base, _DISP_CH)])
    pltpu.async_copy(hidden_hbm.at[idx_b], rows_v, sem).wait()
    pltpu.sync_copy(rows_v, xs_hbm.at[pl.ds(base + _DISP_CH, _DISP_CH)])


@functools.partial(
    pl.kernel, mesh=_SC_MESH,
    out_type=jax.ShapeDtypeStruct((T, H), jnp.float32),
    scratch_types=[
        pltpu.VMEM((_COMB_CH,), jnp.int32),
        pltpu.VMEM((_COMB_CH,), jnp.int32),
        pltpu.VMEM((_COMB_CH, H), jnp.float32),
        pltpu.VMEM((_COMB_CH, H), jnp.float32),
        pltpu.SemaphoreType.DMA,
        pltpu.SemaphoreType.DMA,
    ],
)
def _sc_combine(ys_hbm, pos0_hbm, pos1_hbm, out_hbm,
                idx0, idx1, buf0, buf1, sem0, sem1):
    wbase = _sc_wid() * _COMB_TPW
    for c in range(2):
        tb = wbase + c * _COMB_CH
        pltpu.sync_copy(pos0_hbm.at[pl.ds(tb, _COMB_CH)], idx0)
        pltpu.sync_copy(pos1_hbm.at[pl.ds(tb, _COMB_CH)], idx1)
        cp0 = pltpu.async_copy(ys_hbm.at[idx0], buf0, sem0)
        cp1 = pltpu.async_copy(ys_hbm.at[idx1], buf1, sem1)
        cp0.wait()
        cp1.wait()

        def _add(i, _):
            r = i >> 4
            c0 = pl.multiple_of((i & 15) << 6, 64)
            for u in range(4):
                sl = pl.ds(c0 + u * 16, 16)
                buf0[r, sl] = buf0[r, sl] + buf1[r, sl]
            return 0

        lax.fori_loop(0, _COMB_CH * 16, _add, 0, unroll=False)
        pltpu.sync_copy(buf0, out_hbm.at[pl.ds(tb, _COMB_CH)])


def kernel(hidden_states, expert_affinities, expert_index, w_gate, w_up,
           w_down, seq_len):
    row_token, row_weight, block_expert, pair_pos, num_blocks = _routing(
        expert_affinities, expert_index)
    scalars = jnp.concatenate([block_expert, num_blocks[None]])
    xs = _sc_dispatch(hidden_states, row_token)
    ys = _grouped_mlp(xs, w_gate, w_up, w_down, row_weight, scalars)
    out = _sc_combine(ys, pair_pos[:, 0].ravel(), pair_pos[:, 1].ravel())
    return out
